# Initial kernel scaffold; baseline (speedup 1.0000x reference)
import jax
import jax.numpy as jnp
from jax.experimental import pallas as pl


def kernel(x, edge_index, z_in, z_out):
    raise NotImplementedError("write your pallas kernel here")


# speedup vs baseline: 1.3978x; 1.3978x over previous
import functools

import jax
import jax.numpy as jnp
from jax import lax
from jax.experimental import pallas as pl
from jax.experimental.pallas import tpu as pltpu
from jax.experimental.pallas import tpu_sc as plsc

N_NODES = 10000
NODE_DIM = 128
N_EDGES = 320000
MAX_DEG = 512

NC = 2
NS = 16
NW = NC * NS

NPAD = 10240
CPT = NPAD // NS
ROWW = 100
NROWS = N_EDGES // ROWW
RPT = NROWS // NS
FIRE = 10

BLK = 80
NBLK = N_NODES // BLK
MAXB = (NBLK + NW - 1) // NW

_mesh = plsc.VectorSubcoreMesh(core_axis_name="c", subcore_axis_name="s")


@functools.partial(
    pl.kernel,
    out_type=(
        jax.ShapeDtypeStruct((NPAD,), jnp.int32),
        jax.ShapeDtypeStruct((NPAD,), jnp.int32),
    ),
    mesh=_mesh,
    scratch_types=[
        pltpu.VMEM((RPT, ROWW), jnp.int32),
        pltpu.VMEM((112,), jnp.int32),
        pltpu.VMEM((CPT,), jnp.int32),
        pltpu.VMEM_SHARED((NPAD,), jnp.int32),
        pltpu.SemaphoreType.DMA,
    ],
)
def _degrees(edge_hbm, dego_hbm, degi_hbm, idx_v, ones_v, buf_v, hist_sh, sem):
    c = lax.axis_index("c")
    s = lax.axis_index("s")
    for j in range(112 // 16):
        ones_v[pl.ds(j * 16, 16)] = jnp.full((16,), 1, jnp.int32)
    for j in range(CPT // 16):
        buf_v[pl.ds(j * 16, 16)] = jnp.zeros((16,), jnp.int32)
    pltpu.sync_copy(buf_v, hist_sh.at[pl.ds(s * CPT, CPT)])
    pltpu.sync_copy(edge_hbm.at[c, s], idx_v)
    plsc.subcore_barrier()

    def fire_drain(g, carry):
        hs = []
        for i in range(FIRE):
            j = g * FIRE + i
            hs.append(
                pltpu.async_copy(
                    ones_v.at[pl.ds(0, ROWW)],
                    hist_sh.at[idx_v.at[j]],
                    sem,
                    add=True,
                )
            )
        for h in hs:
            h.wait()
        return carry

    lax.fori_loop(0, RPT // FIRE, fire_drain, 0)
    plsc.subcore_barrier()

    pltpu.sync_copy(hist_sh.at[pl.ds(s * CPT, CPT)], buf_v)
    for j in range(CPT // 16):
        sl = pl.ds(j * 16, 16)
        buf_v[sl] = jnp.minimum(buf_v[sl], MAX_DEG - 1)

    @pl.when(c == 0)
    def _():
        pltpu.sync_copy(buf_v, dego_hbm.at[pl.ds(s * CPT, CPT)])

    @pl.when(c == 1)
    def _():
        pltpu.sync_copy(buf_v, degi_hbm.at[pl.ds(s * CPT, CPT)])


@functools.partial(
    pl.kernel,
    out_type=jax.ShapeDtypeStruct((N_NODES, NODE_DIM), jnp.float32),
    mesh=_mesh,
    scratch_types=[
        pltpu.VMEM((BLK,), jnp.int32),
        pltpu.VMEM((BLK,), jnp.int32),
        pltpu.VMEM((BLK, NODE_DIM), jnp.float32),
        pltpu.VMEM((BLK, NODE_DIM), jnp.float32),
        pltpu.VMEM((BLK, NODE_DIM), jnp.float32),
        pltpu.SemaphoreType.DMA,
        pltpu.SemaphoreType.DMA,
        pltpu.SemaphoreType.DMA,
    ],
)
def _encode(x_hbm, dego_hbm, degi_hbm, zin_hbm, zout_hbm, out_hbm,
            degi_v, dego_v, acc_v, zi_v, zo_v, semx, semi, semo):
    c = lax.axis_index("c")
    s = lax.axis_index("s")
    w = s * NC + c
    for k in range(MAXB):
        b = k * NW + w

        @pl.when(b < NBLK)
        def _():
            base = b * BLK
            hx = pltpu.async_copy(x_hbm.at[pl.ds(base, BLK), :], acc_v, semx)
            pltpu.sync_copy(degi_hbm.at[pl.ds(base, BLK)], degi_v)
            pltpu.sync_copy(dego_hbm.at[pl.ds(base, BLK)], dego_v)
            hi = pltpu.async_copy(zin_hbm.at[degi_v], zi_v, semi)
            ho = pltpu.async_copy(zout_hbm.at[dego_v], zo_v, semo)
            hx.wait()
            hi.wait()
            ho.wait()

            def row_add(r, carry):
                for jj in range(NODE_DIM // 16):
                    sl = pl.ds(jj * 16, 16)
                    acc_v[r, sl] = acc_v[r, sl] + zi_v[r, sl] + zo_v[r, sl]
                return carry

            lax.fori_loop(0, BLK, row_add, 0)
            pltpu.sync_copy(acc_v, out_hbm.at[pl.ds(base, BLK), :])


def kernel(x, edge_index, z_in, z_out):
    ei = edge_index.astype(jnp.int32).reshape(2, NS, RPT, ROWW)
    dego, degi = _degrees(ei)
    return _encode(x, dego, degi, z_in, z_out)
